# baseline (speedup 1.0000x reference)
import jax
import jax.numpy as jnp
from jax.experimental import pallas as pl
from jax.experimental.pallas import tpu as pltpu


def _conv_kernel(ho, wo, xp_ref, w_ref, b_ref, o_ref):
    phases = [xp_ref[0, p] for p in range(4)]
    taps = []
    for kh in range(3):
        for kw in range(3):
            ph, pw = kh % 2, kw % 2
            di, dj = kh // 2, kw // 2
            t = phases[ph * 2 + pw][di:di + ho, dj:dj + wo, :]
            taps.append(t.reshape(ho * wo, t.shape[-1]))
    patches = jnp.concatenate(taps, axis=1)
    acc = jnp.dot(patches, w_ref[...],
                  preferred_element_type=jnp.float32)
    o_ref[0] = jnp.transpose(acc) + b_ref[...]


def kernel(x, w, b):
    B, C, H, W = x.shape
    Co = w.shape[0]
    Ho, Wo = H // 2, W // 2
    N = Ho * Wo
    Hh, Wh = Ho + 1, Wo + 1

    x_pad = jnp.pad(x, ((0, 0), (0, 0), (0, 2), (0, 2)))
    xp = (x_pad.reshape(B, C, Hh, 2, Wh, 2)
          .transpose(0, 3, 5, 2, 4, 1)
          .reshape(B, 4, Hh, Wh, C)
          .astype(jnp.bfloat16))

    w_mat = jnp.transpose(w, (2, 3, 1, 0)).reshape(9 * C, Co).astype(jnp.bfloat16)
    b_col = b.reshape(Co, 1)

    out = pl.pallas_call(
        lambda *refs: _conv_kernel(Ho, Wo, *refs),
        out_shape=jax.ShapeDtypeStruct((B, Co, N), jnp.float32),
        grid=(B,),
        in_specs=[
            pl.BlockSpec((1, 4, Hh, Wh, C), lambda i: (i, 0, 0, 0, 0)),
            pl.BlockSpec((9 * C, Co), lambda i: (0, 0)),
            pl.BlockSpec((Co, 1), lambda i: (0, 0)),
        ],
        out_specs=pl.BlockSpec((1, Co, N), lambda i: (i, 0, 0)),
        compiler_params=pltpu.CompilerParams(
            dimension_semantics=("parallel",),
            vmem_limit_bytes=64 * 1024 * 1024,
        ),
    )(xp, w_mat, b_col)

    return out.reshape(B, Co, Ho, Wo)


# speedup vs baseline: 1.0722x; 1.0722x over previous
import jax
import jax.numpy as jnp
from jax.experimental import pallas as pl
from jax.experimental.pallas import tpu as pltpu


def _conv_kernel(ho, wo, x_ref, w_ref, b_ref, o_ref):
    c = x_ref.shape[1]
    xt = jnp.transpose(x_ref[0])
    x5 = xt.reshape(ho, 2, wo, 2, c)

    phase = [[x5[:, p, :, q, :].astype(jnp.bfloat16) for q in range(2)]
             for p in range(2)]

    zrow = jnp.zeros((1, wo, c), jnp.bfloat16)
    zcol = jnp.zeros((ho, 1, c), jnp.bfloat16)
    taps = []
    for kh in range(3):
        for kw in range(3):
            t = phase[kh % 2][kw % 2]
            if kh // 2:
                t = jnp.concatenate([t[1:], zrow], axis=0)
            if kw // 2:
                t = jnp.concatenate([t[:, 1:], zcol], axis=1)
            taps.append(t.reshape(ho * wo, c))
    patches = jnp.concatenate(taps, axis=1)
    acc = jnp.dot(patches, w_ref[...], preferred_element_type=jnp.float32)
    o_ref[0] = acc + b_ref[...]


def kernel(x, w, b):
    B, C, H, W = x.shape
    Co = w.shape[0]
    Ho, Wo = H // 2, W // 2
    N = Ho * Wo

    x_flat = x.reshape(B, C, H * W)

    w_mat = jnp.transpose(w, (2, 3, 1, 0)).reshape(9 * C, Co).astype(jnp.bfloat16)
    b_row = b.reshape(1, Co)

    out = pl.pallas_call(
        lambda *refs: _conv_kernel(Ho, Wo, *refs),
        out_shape=jax.ShapeDtypeStruct((B, N, Co), jnp.float32),
        grid=(B,),
        in_specs=[
            pl.BlockSpec((1, C, H * W), lambda i: (i, 0, 0)),
            pl.BlockSpec((9 * C, Co), lambda i: (0, 0)),
            pl.BlockSpec((1, Co), lambda i: (0, 0)),
        ],
        out_specs=pl.BlockSpec((1, N, Co), lambda i: (i, 0, 0)),
        compiler_params=pltpu.CompilerParams(
            dimension_semantics=("parallel",),
            vmem_limit_bytes=64 * 1024 * 1024,
        ),
    )(x_flat, w_mat, b_row)

    return out.transpose(0, 2, 1).reshape(B, Co, Ho, Wo)


# speedup vs baseline: 1.2480x; 1.1640x over previous
import jax
import jax.numpy as jnp
from jax.experimental import pallas as pl
from jax.experimental.pallas import tpu as pltpu


def _conv_kernel(ho, wo, x_ref, w_ref, b_ref, o_ref):
    c = x_ref.shape[1]
    xt = jnp.transpose(x_ref[0])
    x5 = xt.astype(jnp.bfloat16).reshape(ho, 2, wo, 2, c)

    phase = [[x5[:, p, :, q, :] for q in range(2)]
             for p in range(2)]

    zrow = jnp.zeros((1, wo, c), jnp.bfloat16)
    zcol = jnp.zeros((ho, 1, c), jnp.bfloat16)
    taps = []
    for kh in range(3):
        for kw in range(3):
            t = phase[kh % 2][kw % 2]
            if kh // 2:
                t = jnp.concatenate([t[1:], zrow], axis=0)
            if kw // 2:
                t = jnp.concatenate([t[:, 1:], zcol], axis=1)
            taps.append(t.reshape(ho * wo, c))
    patches = jnp.concatenate(taps, axis=1)
    acc = jnp.dot(patches, w_ref[...], preferred_element_type=jnp.float32)
    o_ref[0] = acc + b_ref[...]


def kernel(x, w, b):
    B, C, H, W = x.shape
    Co = w.shape[0]
    Ho, Wo = H // 2, W // 2
    N = Ho * Wo

    x_flat = x.reshape(B, C, H * W)

    w_mat = jnp.transpose(w, (2, 3, 1, 0)).reshape(9 * C, Co).astype(jnp.bfloat16)
    b_row = b.reshape(1, Co)

    out = pl.pallas_call(
        lambda *refs: _conv_kernel(Ho, Wo, *refs),
        out_shape=jax.ShapeDtypeStruct((B, N, Co), jnp.float32),
        grid=(2, B // 2),
        in_specs=[
            pl.BlockSpec((1, C, H * W), lambda ci, i: (ci * (B // 2) + i, 0, 0)),
            pl.BlockSpec((9 * C, Co), lambda ci, i: (0, 0)),
            pl.BlockSpec((1, Co), lambda ci, i: (0, 0)),
        ],
        out_specs=pl.BlockSpec((1, N, Co), lambda ci, i: (ci * (B // 2) + i, 0, 0)),
        compiler_params=pltpu.CompilerParams(
            dimension_semantics=("parallel", "arbitrary"),
            vmem_limit_bytes=64 * 1024 * 1024,
        ),
    )(x_flat, w_mat, b_row)

    return out.transpose(0, 2, 1).reshape(B, Co, Ho, Wo)
